# baseline (speedup 1.0000x reference)
import jax
import jax.numpy as jnp
from jax.experimental import pallas as pl

_R = 128
_N = 100000
_RB = 16


def _argmax_kernel(x_ref, out_ref):
    out_ref[...] = jnp.argmax(x_ref[...], axis=1).astype(jnp.int32)[:, None]


def kernel(probs):
    out = pl.pallas_call(
        _argmax_kernel,
        grid=(_R // _RB,),
        in_specs=[pl.BlockSpec((_RB, _N), lambda i: (i, 0))],
        out_specs=pl.BlockSpec((_RB, 1), lambda i: (i, 0)),
        out_shape=jax.ShapeDtypeStruct((_R, 1), jnp.int32),
    )(probs)
    return out[:, 0]


# speedup vs baseline: 1.0379x; 1.0379x over previous
import jax
import jax.numpy as jnp
from jax.experimental import pallas as pl
from jax.experimental.pallas import tpu as pltpu

_R = 128
_N = 100000
_RB = 16
_G = _R // _RB
_NBUF = 4


def _slab_copy(x_hbm, buf, sem, i, slot):
    return pltpu.make_async_copy(
        x_hbm.at[pl.ds(i * _RB, _RB), :], buf.at[slot], sem.at[slot]
    )


def _argmax_kernel(x_hbm, out_ref, buf, sem):
    for s in range(min(_NBUF, _G)):
        _slab_copy(x_hbm, buf, sem, s, s).start()
    for i in range(_G):
        slot = i % _NBUF
        _slab_copy(x_hbm, buf, sem, i, slot).wait()
        out_ref[pl.ds(i * _RB, _RB), :] = (
            jnp.argmax(buf[slot], axis=1).astype(jnp.int32)[:, None]
        )
        nxt = i + _NBUF
        if nxt < _G:
            _slab_copy(x_hbm, buf, sem, nxt, nxt % _NBUF).start()


def kernel(probs):
    out = pl.pallas_call(
        _argmax_kernel,
        in_specs=[pl.BlockSpec(memory_space=pltpu.MemorySpace.HBM)],
        out_specs=pl.BlockSpec(memory_space=pltpu.MemorySpace.VMEM),
        out_shape=jax.ShapeDtypeStruct((_R, 1), jnp.int32),
        scratch_shapes=[
            pltpu.VMEM((_NBUF, _RB, _N), jnp.float32),
            pltpu.SemaphoreType.DMA((_NBUF,)),
        ],
    )(probs)
    return out[:, 0]
